# baseline (speedup 1.0000x reference)
import functools

import jax
import jax.numpy as jnp
from jax import lax
from jax.experimental import pallas as pl
from jax.experimental.pallas import tpu as pltpu
from jax.experimental.pallas import tpu_sc as plsc

NC = 2
NS = 16
NW = NC * NS
K = 128
NDUMP = 32


def _deg_hist_sc(dst2d, zeros1, ones, n_pad, ncw):
    rt = n_pad // NS
    kd = 8
    mesh = plsc.VectorSubcoreMesh(core_axis_name="c", subcore_axis_name="s")

    @functools.partial(
        pl.kernel, mesh=mesh,
        out_type=jax.ShapeDtypeStruct((NC * n_pad,), jnp.float32),
        scratch_types=[
            pltpu.VMEM((ncw, K), jnp.int32),
            pltpu.VMEM((K,), jnp.float32),
            pltpu.VMEM((rt,), jnp.float32),
            pltpu.VMEM_SHARED((n_pad,), jnp.float32),
            pltpu.SemaphoreType.DMA,
        ],
    )
    def k(dst_hbm, z_hbm, ones_hbm, out_hbm, dstv, ones_v, buf_v, acc_sh, sem):
        c = lax.axis_index("c")
        s = lax.axis_index("s")
        w = s * NC + c
        pltpu.sync_copy(z_hbm, buf_v)
        pltpu.sync_copy(buf_v, acc_sh.at[pl.ds(s * rt, rt)])
        pltpu.sync_copy(ones_hbm, ones_v)
        pltpu.sync_copy(dst_hbm.at[pl.ds(w * ncw, ncw)], dstv)
        plsc.subcore_barrier()

        def body(jj, carry):
            for b in range(kd):
                pltpu.async_copy(ones_v, acc_sh.at[dstv.at[jj * kd + b]],
                                 sem, add=True)
            for b in range(kd):
                pltpu.make_async_copy(ones_v, acc_sh.at[dstv.at[0]], sem).wait()
            return carry

        lax.fori_loop(0, ncw // kd, body, 0)
        plsc.subcore_barrier()
        pltpu.sync_copy(acc_sh.at[pl.ds(s * rt, rt)], buf_v)
        pltpu.sync_copy(buf_v, out_hbm.at[pl.ds(c * n_pad + s * rt, rt)])

    return k(dst2d, zeros1, ones)


def _segsum_sc(src2d, dst2d, y, zeros2, n_pad, d, ncw):
    half = n_pad // NC
    rt = half // NS
    ncw2 = ncw // 2
    nout2 = ncw2 // 4
    mesh = plsc.VectorSubcoreMesh(core_axis_name="c", subcore_axis_name="s")

    @functools.partial(
        pl.kernel, mesh=mesh,
        out_type=jax.ShapeDtypeStruct((n_pad, d), jnp.float32),
        scratch_types=[
            pltpu.VMEM((ncw2, K), jnp.int32),
            pltpu.VMEM((ncw2, K), jnp.int32),
            pltpu.VMEM((K, d), jnp.float32),
            pltpu.VMEM((K, d), jnp.float32),
            pltpu.VMEM((K, d), jnp.float32),
            pltpu.VMEM((K, d), jnp.float32),
            pltpu.VMEM_SHARED((half + NDUMP, d), jnp.float32),
            pltpu.SemaphoreType.DMA,
            pltpu.SemaphoreType.DMA,
            pltpu.SemaphoreType.DMA,
            pltpu.SemaphoreType.DMA,
            pltpu.SemaphoreType.DMA,
            pltpu.SemaphoreType.DMA,
            pltpu.SemaphoreType.DMA,
            pltpu.SemaphoreType.DMA,
        ],
    )
    def k(src_hbm, dst_hbm, y_hbm, z_hbm, out_hbm, srcv, dstv,
          r0, r1, r2, r3, acc_sh, g0, g1, g2, g3, s0, s1, s2, s3):
        rows = (r0, r1, r2, r3)
        gsem = (g0, g1, g2, g3)
        ssem = (s0, s1, s2, s3)
        c = lax.axis_index("c")
        s = lax.axis_index("s")
        base = c * half

        pltpu.sync_copy(z_hbm, r0)
        zoff = 0
        for sz in (K, K, rt - 2 * K):
            pltpu.sync_copy(r0.at[pl.ds(0, sz)],
                            acc_sh.at[pl.ds(s * rt + zoff, sz)])
            zoff += sz

        @pl.when(s == 0)
        def _():
            pltpu.sync_copy(r0.at[pl.ds(0, NDUMP)], acc_sh.at[pl.ds(half, NDUMP)])

        plsc.subcore_barrier()

        for h in range(2):
            hb = s * (2 * ncw2) + h * ncw2
            pltpu.sync_copy(src_hbm.at[pl.ds(hb, ncw2)], srcv)
            pltpu.sync_copy(dst_hbm.at[pl.ds(hb, ncw2)], dstv)
            pltpu.async_copy(y_hbm.at[srcv.at[0]], r0, g0)
            pltpu.async_copy(y_hbm.at[srcv.at[1]], r1, g1)

            def remap(jc, carry):
                for li in range(K // 16):
                    v = dstv[jc, pl.ds(li * 16, 16)]
                    v2 = v - base
                    inb = jnp.logical_and(v2 >= 0, v2 < half)
                    dump = half + jnp.bitwise_and(v, NDUMP - 1)
                    dstv[jc, pl.ds(li * 16, 16)] = jnp.where(inb, v2, dump)
                return carry

            lax.fori_loop(0, ncw2, remap, 0)

            def body(jj, carry):
                for b in range(4):
                    j = jj * 4 + b
                    nb = (b + 2) % 4
                    pltpu.make_async_copy(y_hbm.at[srcv.at[0]], rows[b],
                                          gsem[b]).wait()
                    pltpu.async_copy(rows[b], acc_sh.at[dstv.at[j]],
                                     ssem[b], add=True)
                    if b < 2:
                        @pl.when(jj > 0)
                        def _():
                            pltpu.make_async_copy(
                                rows[nb], acc_sh.at[dstv.at[0]],
                                ssem[nb]).wait()
                        pltpu.async_copy(y_hbm.at[srcv.at[j + 2]], rows[nb],
                                         gsem[nb])
                    else:
                        @pl.when(jj < nout2 - 1)
                        def _():
                            pltpu.make_async_copy(
                                rows[nb], acc_sh.at[dstv.at[0]],
                                ssem[nb]).wait()
                            pltpu.async_copy(y_hbm.at[srcv.at[j + 2]], rows[nb],
                                             gsem[nb])
                return carry

            lax.fori_loop(0, nout2, body, 0)
            pltpu.make_async_copy(r0, acc_sh.at[dstv.at[0]], s0).wait()
            pltpu.make_async_copy(r1, acc_sh.at[dstv.at[0]], s1).wait()
            pltpu.make_async_copy(r2, acc_sh.at[dstv.at[0]], s2).wait()
            pltpu.make_async_copy(r3, acc_sh.at[dstv.at[0]], s3).wait()

        plsc.subcore_barrier()

        off = 0
        for sz in (K, K, rt - 2 * K):
            pltpu.sync_copy(acc_sh.at[pl.ds(s * rt + off, sz)], r0.at[pl.ds(0, sz)])
            pltpu.sync_copy(r0.at[pl.ds(0, sz)],
                            out_hbm.at[pl.ds(base + s * rt + off, sz)])
            off += sz

    return k(src2d, dst2d, y, zeros2)


def _y_tc(onehot, w_mat, p0, p1, n, d, rows):
    def body(x_ref, w_ref, p0_ref, p1_ref, y_ref):
        xw = jnp.dot(x_ref[...], w_ref[...], preferred_element_type=jnp.float32)
        dinv = lax.rsqrt(p0_ref[...] + p1_ref[...] + 1.0)
        y_ref[...] = xw * dinv

    return pl.pallas_call(
        body,
        grid=(n // rows,),
        in_specs=[
            pl.BlockSpec((rows, d), lambda i: (i, 0)),
            pl.BlockSpec((d, d), lambda i: (0, 0)),
            pl.BlockSpec((rows, 1), lambda i: (i, 0)),
            pl.BlockSpec((rows, 1), lambda i: (i, 0)),
        ],
        out_specs=pl.BlockSpec((rows, d), lambda i: (i, 0)),
        out_shape=jax.ShapeDtypeStruct((n, d), jnp.float32),
    )(onehot, w_mat, p0, p1)


def _finish_tc(acc, y, p0, p1, bias, n, d, rows):
    def body(a_ref, y_ref, p0_ref, p1_ref, b_ref, o_ref):
        dinv = lax.rsqrt(p0_ref[...] + p1_ref[...] + 1.0)
        z = (a_ref[...] + y_ref[...]) * dinv + b_ref[...]
        z = jnp.maximum(z, 0.0)
        m = jnp.max(z, axis=1, keepdims=True)
        ez = jnp.exp(z - m)
        o_ref[...] = ez / jnp.sum(ez, axis=1, keepdims=True)

    return pl.pallas_call(
        body,
        grid=(n // rows,),
        in_specs=[
            pl.BlockSpec((rows, d), lambda i: (i, 0)),
            pl.BlockSpec((rows, d), lambda i: (i, 0)),
            pl.BlockSpec((rows, 1), lambda i: (i, 0)),
            pl.BlockSpec((rows, 1), lambda i: (i, 0)),
            pl.BlockSpec((1, d), lambda i: (0, 0)),
        ],
        out_specs=pl.BlockSpec((rows, d), lambda i: (i, 0)),
        out_shape=jax.ShapeDtypeStruct((n, d), jnp.float32),
    )(acc, y, p0, p1, bias)


def kernel(onehot, edge_index, W, b):
    n, d = onehot.shape
    e = edge_index.shape[1]
    n_pad = -(-n // (NW * 8)) * NW * 8
    ncw_deg = -(-(-(-e // (NW * K))) // 8) * 8
    ncw_seg = ncw_deg * NC
    e_pad = NW * ncw_deg * K
    src = edge_index[0]
    dst = edge_index[1]
    src2d = jnp.concatenate(
        [src, jnp.zeros((e_pad - e,), jnp.int32)]).reshape(NW * ncw_deg, K)
    dst2d = jnp.concatenate(
        [dst, jnp.full((e_pad - e,), n_pad - 1, jnp.int32)]).reshape(
            NW * ncw_deg, K)

    zeros1 = jnp.zeros((n_pad // NS,), jnp.float32)
    zeros2 = jnp.zeros((K, d), jnp.float32)
    ones = jnp.ones((K,), jnp.float32)

    degp = _deg_hist_sc(dst2d, zeros1, ones, n_pad, ncw_deg)
    p0 = degp[:n][:, None]
    p1 = degp[n_pad:n_pad + n][:, None]
    y = _y_tc(onehot, W, p0, p1, n, d, rows=1000)
    accp = _segsum_sc(src2d, dst2d, y, zeros2, n_pad, d, ncw_seg)
    return _finish_tc(accp[:n], y, p0, p1, b.reshape(1, d), n, d, rows=1000)


# speedup vs baseline: 1.7578x; 1.7578x over previous
import functools

import jax
import jax.numpy as jnp
from jax import lax
from jax.experimental import pallas as pl
from jax.experimental.pallas import tpu as pltpu
from jax.experimental.pallas import tpu_sc as plsc

NC = 2
NS = 16
NW = NC * NS
K = 128
K2 = 64


def _deg_hist_sc(dst2d, zeros1, ones, n_pad, ncw):
    rt = n_pad // NS
    kd = 8
    mesh = plsc.VectorSubcoreMesh(core_axis_name="c", subcore_axis_name="s")

    @functools.partial(
        pl.kernel, mesh=mesh,
        out_type=jax.ShapeDtypeStruct((NC * n_pad,), jnp.float32),
        scratch_types=[
            pltpu.VMEM((ncw, K), jnp.int32),
            pltpu.VMEM((K,), jnp.float32),
            pltpu.VMEM((rt,), jnp.float32),
            pltpu.VMEM_SHARED((n_pad,), jnp.float32),
            pltpu.SemaphoreType.DMA,
        ],
    )
    def k(dst_hbm, z_hbm, ones_hbm, out_hbm, dstv, ones_v, buf_v, acc_sh, sem):
        c = lax.axis_index("c")
        s = lax.axis_index("s")
        w = s * NC + c
        pltpu.sync_copy(z_hbm, buf_v)
        pltpu.sync_copy(buf_v, acc_sh.at[pl.ds(s * rt, rt)])
        pltpu.sync_copy(ones_hbm, ones_v)
        pltpu.sync_copy(dst_hbm.at[pl.ds(w * ncw, ncw)], dstv)
        plsc.subcore_barrier()

        def body(jj, carry):
            for b in range(kd):
                pltpu.async_copy(ones_v, acc_sh.at[dstv.at[jj * kd + b]],
                                 sem, add=True)
            for b in range(kd):
                pltpu.make_async_copy(ones_v, acc_sh.at[dstv.at[0]], sem).wait()
            return carry

        lax.fori_loop(0, ncw // kd, body, 0)
        plsc.subcore_barrier()
        pltpu.sync_copy(acc_sh.at[pl.ds(s * rt, rt)], buf_v)
        pltpu.sync_copy(buf_v, out_hbm.at[pl.ds(c * n_pad + s * rt, rt)])

    return k(dst2d, zeros1, ones)


def _segsum_sc(src2d, dst2d, y, zeros2, n_pad, d, ncw):
    rt = n_pad // NS
    ncw2 = ncw // 4
    nout2 = ncw2 // 4
    mesh = plsc.VectorSubcoreMesh(core_axis_name="c", subcore_axis_name="s")

    @functools.partial(
        pl.kernel, mesh=mesh,
        out_type=jax.ShapeDtypeStruct((NC * n_pad, d), jnp.float32),
        scratch_types=[
            pltpu.VMEM((ncw2, K2), jnp.int32),
            pltpu.VMEM((ncw2, K2), jnp.int32),
            pltpu.VMEM((K2, d), jnp.float32),
            pltpu.VMEM((K2, d), jnp.float32),
            pltpu.VMEM((K2, d), jnp.float32),
            pltpu.VMEM((K2, d), jnp.float32),
            pltpu.VMEM_SHARED((n_pad, d), jnp.float32),
            pltpu.SemaphoreType.DMA,
            pltpu.SemaphoreType.DMA,
            pltpu.SemaphoreType.DMA,
            pltpu.SemaphoreType.DMA,
            pltpu.SemaphoreType.DMA,
            pltpu.SemaphoreType.DMA,
            pltpu.SemaphoreType.DMA,
            pltpu.SemaphoreType.DMA,
        ],
    )
    def k(src_hbm, dst_hbm, y_hbm, z_hbm, out_hbm, srcv, dstv,
          r0, r1, r2, r3, acc_sh, g0, g1, g2, g3, s0, s1, s2, s3):
        rows = (r0, r1, r2, r3)
        gsem = (g0, g1, g2, g3)
        ssem = (s0, s1, s2, s3)
        c = lax.axis_index("c")
        s = lax.axis_index("s")
        w = s * NC + c

        pltpu.sync_copy(z_hbm, r0)
        for jz in range(rt // K2):
            pltpu.sync_copy(r0, acc_sh.at[pl.ds(s * rt + jz * K2, K2)])
        plsc.subcore_barrier()

        for h in range(4):
            hb = w * ncw + h * ncw2
            pltpu.sync_copy(src_hbm.at[pl.ds(hb, ncw2)], srcv)
            pltpu.sync_copy(dst_hbm.at[pl.ds(hb, ncw2)], dstv)
            pltpu.async_copy(y_hbm.at[srcv.at[0]], r0, g0)
            pltpu.async_copy(y_hbm.at[srcv.at[1]], r1, g1)

            def body(jj, carry):
                for b in range(4):
                    j = jj * 4 + b
                    nb = (b + 2) % 4
                    pltpu.make_async_copy(y_hbm.at[srcv.at[0]], rows[b],
                                          gsem[b]).wait()
                    pltpu.async_copy(rows[b], acc_sh.at[dstv.at[j]],
                                     ssem[b], add=True)
                    if b < 2:
                        @pl.when(jj > 0)
                        def _():
                            pltpu.make_async_copy(
                                rows[nb], acc_sh.at[dstv.at[0]],
                                ssem[nb]).wait()
                        pltpu.async_copy(y_hbm.at[srcv.at[j + 2]], rows[nb],
                                         gsem[nb])
                    else:
                        @pl.when(jj < nout2 - 1)
                        def _():
                            pltpu.make_async_copy(
                                rows[nb], acc_sh.at[dstv.at[0]],
                                ssem[nb]).wait()
                            pltpu.async_copy(y_hbm.at[srcv.at[j + 2]], rows[nb],
                                             gsem[nb])
                return carry

            lax.fori_loop(0, nout2, body, 0)
            pltpu.make_async_copy(r0, acc_sh.at[dstv.at[0]], s0).wait()
            pltpu.make_async_copy(r1, acc_sh.at[dstv.at[0]], s1).wait()
            pltpu.make_async_copy(r2, acc_sh.at[dstv.at[0]], s2).wait()
            pltpu.make_async_copy(r3, acc_sh.at[dstv.at[0]], s3).wait()

        plsc.subcore_barrier()

        for jz in range(rt // K2):
            pltpu.sync_copy(acc_sh.at[pl.ds(s * rt + jz * K2, K2)], r0)
            pltpu.sync_copy(
                r0, out_hbm.at[pl.ds(c * n_pad + s * rt + jz * K2, K2)])

    return k(src2d, dst2d, y, zeros2)


def _y_tc(onehot, w_mat, p0, p1, n, d, rows):
    def body(x_ref, w_ref, p0_ref, p1_ref, y_ref):
        xw = jnp.dot(x_ref[...], w_ref[...], preferred_element_type=jnp.float32)
        dinv = lax.rsqrt(p0_ref[...] + p1_ref[...] + 1.0)
        y_ref[...] = xw * dinv

    return pl.pallas_call(
        body,
        grid=(n // rows,),
        in_specs=[
            pl.BlockSpec((rows, d), lambda i: (i, 0)),
            pl.BlockSpec((d, d), lambda i: (0, 0)),
            pl.BlockSpec((rows, 1), lambda i: (i, 0)),
            pl.BlockSpec((rows, 1), lambda i: (i, 0)),
        ],
        out_specs=pl.BlockSpec((rows, d), lambda i: (i, 0)),
        out_shape=jax.ShapeDtypeStruct((n, d), jnp.float32),
    )(onehot, w_mat, p0, p1)


def _finish_tc(a0, a1, y, p0, p1, bias, n, d, rows):
    def body(a0_ref, a1_ref, y_ref, p0_ref, p1_ref, b_ref, o_ref):
        dinv = lax.rsqrt(p0_ref[...] + p1_ref[...] + 1.0)
        z = (a0_ref[...] + a1_ref[...] + y_ref[...]) * dinv + b_ref[...]
        z = jnp.maximum(z, 0.0)
        m = jnp.max(z, axis=1, keepdims=True)
        ez = jnp.exp(z - m)
        o_ref[...] = ez / jnp.sum(ez, axis=1, keepdims=True)

    return pl.pallas_call(
        body,
        grid=(n // rows,),
        in_specs=[
            pl.BlockSpec((rows, d), lambda i: (i, 0)),
            pl.BlockSpec((rows, d), lambda i: (i, 0)),
            pl.BlockSpec((rows, d), lambda i: (i, 0)),
            pl.BlockSpec((rows, 1), lambda i: (i, 0)),
            pl.BlockSpec((rows, 1), lambda i: (i, 0)),
            pl.BlockSpec((1, d), lambda i: (0, 0)),
        ],
        out_specs=pl.BlockSpec((rows, d), lambda i: (i, 0)),
        out_shape=jax.ShapeDtypeStruct((n, d), jnp.float32),
    )(a0, a1, y, p0, p1, bias)


def kernel(onehot, edge_index, W, b):
    n, d = onehot.shape
    e = edge_index.shape[1]
    n_pad = -(-n // (NW * 8)) * NW * 8
    ncw_deg = -(-(-(-e // (NW * K))) // 8) * 8
    ncw_seg = ncw_deg * (K // K2)
    e_pad = NW * ncw_deg * K
    src = edge_index[0]
    dst = edge_index[1]
    src_p = jnp.concatenate([src, jnp.zeros((e_pad - e,), jnp.int32)])
    dst_p = jnp.concatenate([dst, jnp.full((e_pad - e,), n_pad - 1, jnp.int32)])

    zeros1 = jnp.zeros((n_pad // NS,), jnp.float32)
    zeros2 = jnp.zeros((K2, d), jnp.float32)
    ones = jnp.ones((K,), jnp.float32)

    degp = _deg_hist_sc(dst_p.reshape(NW * ncw_deg, K), zeros1, ones,
                        n_pad, ncw_deg)
    p0 = degp[:n][:, None]
    p1 = degp[n_pad:n_pad + n][:, None]
    y = _y_tc(onehot, W, p0, p1, n, d, rows=1000)
    accp = _segsum_sc(src_p.reshape(NW * ncw_seg, K2),
                      dst_p.reshape(NW * ncw_seg, K2),
                      y, zeros2, n_pad, d, ncw_seg)
    return _finish_tc(accp[:n], accp[n_pad:n_pad + n], y, p0, p1,
                      b.reshape(1, d), n, d, rows=1000)


# speedup vs baseline: 1.8350x; 1.0439x over previous
import functools

import jax
import jax.numpy as jnp
from jax import lax
from jax.experimental import pallas as pl
from jax.experimental.pallas import tpu as pltpu
from jax.experimental.pallas import tpu_sc as plsc

NC = 2
NS = 16
NW = NC * NS
K = 128
K2 = 64
FAST_CORE = 0
FAST_FRAC = 6


def _deg_hist_sc(dst2d, zeros1, ones, n_pad, ncw):
    rt = n_pad // NS
    kd = 8
    mesh = plsc.VectorSubcoreMesh(core_axis_name="c", subcore_axis_name="s")

    @functools.partial(
        pl.kernel, mesh=mesh,
        out_type=jax.ShapeDtypeStruct((NC * n_pad,), jnp.float32),
        scratch_types=[
            pltpu.VMEM((ncw, K), jnp.int32),
            pltpu.VMEM((K,), jnp.float32),
            pltpu.VMEM((rt,), jnp.float32),
            pltpu.VMEM_SHARED((n_pad,), jnp.float32),
            pltpu.SemaphoreType.DMA,
        ],
    )
    def k(dst_hbm, z_hbm, ones_hbm, out_hbm, dstv, ones_v, buf_v, acc_sh, sem):
        c = lax.axis_index("c")
        s = lax.axis_index("s")
        w = s * NC + c
        pltpu.sync_copy(z_hbm, buf_v)
        pltpu.sync_copy(buf_v, acc_sh.at[pl.ds(s * rt, rt)])
        pltpu.sync_copy(ones_hbm, ones_v)
        pltpu.sync_copy(dst_hbm.at[pl.ds(w * ncw, ncw)], dstv)
        plsc.subcore_barrier()

        def body(jj, carry):
            for b in range(kd):
                pltpu.async_copy(ones_v, acc_sh.at[dstv.at[jj * kd + b]],
                                 sem, add=True)
            for b in range(kd):
                pltpu.make_async_copy(ones_v, acc_sh.at[dstv.at[0]], sem).wait()
            return carry

        lax.fori_loop(0, ncw // kd, body, 0)
        plsc.subcore_barrier()
        pltpu.sync_copy(acc_sh.at[pl.ds(s * rt, rt)], buf_v)
        pltpu.sync_copy(buf_v, out_hbm.at[pl.ds(c * n_pad + s * rt, rt)])

    return k(dst2d, zeros1, ones)


def _segsum_sc(src2d, dst2d, y, zeros2, n_pad, d, nchunks):
    rt = n_pad // NS
    per_pair = nchunks // NS
    ncw2 = per_pair // 8
    nout2 = ncw2 // 4
    ncw_fast = FAST_FRAC * ncw2
    ncw_slow = (8 - FAST_FRAC) * ncw2
    mesh = plsc.VectorSubcoreMesh(core_axis_name="c", subcore_axis_name="s")

    @functools.partial(
        pl.kernel, mesh=mesh,
        out_type=jax.ShapeDtypeStruct((NC * n_pad, d), jnp.float32),
        scratch_types=[
            pltpu.VMEM((ncw2, K2), jnp.int32),
            pltpu.VMEM((ncw2, K2), jnp.int32),
            pltpu.VMEM((K2, d), jnp.float32),
            pltpu.VMEM((K2, d), jnp.float32),
            pltpu.VMEM((K2, d), jnp.float32),
            pltpu.VMEM((K2, d), jnp.float32),
            pltpu.VMEM_SHARED((n_pad, d), jnp.float32),
            pltpu.SemaphoreType.DMA,
            pltpu.SemaphoreType.DMA,
            pltpu.SemaphoreType.DMA,
            pltpu.SemaphoreType.DMA,
            pltpu.SemaphoreType.DMA,
            pltpu.SemaphoreType.DMA,
            pltpu.SemaphoreType.DMA,
            pltpu.SemaphoreType.DMA,
        ],
    )
    def k(src_hbm, dst_hbm, y_hbm, z_hbm, out_hbm, srcv, dstv,
          r0, r1, r2, r3, acc_sh, g0, g1, g2, g3, s0, s1, s2, s3):
        rows = (r0, r1, r2, r3)
        gsem = (g0, g1, g2, g3)
        ssem = (s0, s1, s2, s3)
        c = lax.axis_index("c")
        s = lax.axis_index("s")
        is_fast = c == FAST_CORE
        my_nphase = jnp.where(is_fast, FAST_FRAC, 8 - FAST_FRAC)
        base = jnp.where(is_fast, s * ncw_fast,
                         NS * ncw_fast + s * ncw_slow)

        pltpu.sync_copy(z_hbm, r0)
        for jz in range(rt // K2):
            pltpu.sync_copy(r0, acc_sh.at[pl.ds(s * rt + jz * K2, K2)])
        plsc.subcore_barrier()

        for h in range(FAST_FRAC):

            @pl.when(h < my_nphase)
            def _():
                hb = base + h * ncw2
                pltpu.sync_copy(src_hbm.at[pl.ds(hb, ncw2)], srcv)
                pltpu.sync_copy(dst_hbm.at[pl.ds(hb, ncw2)], dstv)
                pltpu.async_copy(y_hbm.at[srcv.at[0]], r0, g0)
                pltpu.async_copy(y_hbm.at[srcv.at[1]], r1, g1)

                def body(jj, carry):
                    for b in range(4):
                        j = jj * 4 + b
                        nb = (b + 2) % 4
                        pltpu.make_async_copy(y_hbm.at[srcv.at[0]], rows[b],
                                              gsem[b]).wait()
                        pltpu.async_copy(rows[b], acc_sh.at[dstv.at[j]],
                                         ssem[b], add=True)
                        if b < 2:
                            @pl.when(jj > 0)
                            def _():
                                pltpu.make_async_copy(
                                    rows[nb], acc_sh.at[dstv.at[0]],
                                    ssem[nb]).wait()
                            pltpu.async_copy(y_hbm.at[srcv.at[j + 2]],
                                             rows[nb], gsem[nb])
                        else:
                            @pl.when(jj < nout2 - 1)
                            def _():
                                pltpu.make_async_copy(
                                    rows[nb], acc_sh.at[dstv.at[0]],
                                    ssem[nb]).wait()
                                pltpu.async_copy(y_hbm.at[srcv.at[j + 2]],
                                                 rows[nb], gsem[nb])
                    return carry

                lax.fori_loop(0, nout2, body, 0)
                pltpu.make_async_copy(r0, acc_sh.at[dstv.at[0]], s0).wait()
                pltpu.make_async_copy(r1, acc_sh.at[dstv.at[0]], s1).wait()
                pltpu.make_async_copy(r2, acc_sh.at[dstv.at[0]], s2).wait()
                pltpu.make_async_copy(r3, acc_sh.at[dstv.at[0]], s3).wait()

        plsc.subcore_barrier()

        for jz in range(rt // K2):
            pltpu.sync_copy(acc_sh.at[pl.ds(s * rt + jz * K2, K2)], r0)
            pltpu.sync_copy(
                r0, out_hbm.at[pl.ds(c * n_pad + s * rt + jz * K2, K2)])

    return k(src2d, dst2d, y, zeros2)


def _y_tc(onehot, w_mat, p0, p1, n, d, rows):
    def body(x_ref, w_ref, p0_ref, p1_ref, y_ref):
        xw = jnp.dot(x_ref[...], w_ref[...], preferred_element_type=jnp.float32)
        dinv = lax.rsqrt(p0_ref[...] + p1_ref[...] + 1.0)
        y_ref[...] = xw * dinv

    return pl.pallas_call(
        body,
        grid=(n // rows,),
        in_specs=[
            pl.BlockSpec((rows, d), lambda i: (i, 0)),
            pl.BlockSpec((d, d), lambda i: (0, 0)),
            pl.BlockSpec((rows, 1), lambda i: (i, 0)),
            pl.BlockSpec((rows, 1), lambda i: (i, 0)),
        ],
        out_specs=pl.BlockSpec((rows, d), lambda i: (i, 0)),
        out_shape=jax.ShapeDtypeStruct((n, d), jnp.float32),
    )(onehot, w_mat, p0, p1)


def _finish_tc(a0, a1, y, p0, p1, bias, n, d, rows):
    def body(a0_ref, a1_ref, y_ref, p0_ref, p1_ref, b_ref, o_ref):
        dinv = lax.rsqrt(p0_ref[...] + p1_ref[...] + 1.0)
        z = (a0_ref[...] + a1_ref[...] + y_ref[...]) * dinv + b_ref[...]
        z = jnp.maximum(z, 0.0)
        m = jnp.max(z, axis=1, keepdims=True)
        ez = jnp.exp(z - m)
        o_ref[...] = ez / jnp.sum(ez, axis=1, keepdims=True)

    return pl.pallas_call(
        body,
        grid=(n // rows,),
        in_specs=[
            pl.BlockSpec((rows, d), lambda i: (i, 0)),
            pl.BlockSpec((rows, d), lambda i: (i, 0)),
            pl.BlockSpec((rows, d), lambda i: (i, 0)),
            pl.BlockSpec((rows, 1), lambda i: (i, 0)),
            pl.BlockSpec((rows, 1), lambda i: (i, 0)),
            pl.BlockSpec((1, d), lambda i: (0, 0)),
        ],
        out_specs=pl.BlockSpec((rows, d), lambda i: (i, 0)),
        out_shape=jax.ShapeDtypeStruct((n, d), jnp.float32),
    )(a0, a1, y, p0, p1, bias)


def kernel(onehot, edge_index, W, b):
    n, d = onehot.shape
    e = edge_index.shape[1]
    n_pad = -(-n // (NW * 8)) * NW * 8
    ncw_deg = -(-(-(-e // (NW * K))) // 8) * 8
    ncw_seg = ncw_deg * (K // K2)
    e_pad = NW * ncw_deg * K
    src = edge_index[0]
    dst = edge_index[1]
    src_p = jnp.concatenate([src, jnp.zeros((e_pad - e,), jnp.int32)])
    dst_p = jnp.concatenate([dst, jnp.full((e_pad - e,), n_pad - 1, jnp.int32)])

    zeros1 = jnp.zeros((n_pad // NS,), jnp.float32)
    zeros2 = jnp.zeros((K2, d), jnp.float32)
    ones = jnp.ones((K,), jnp.float32)

    degp = _deg_hist_sc(dst_p.reshape(NW * ncw_deg, K), zeros1, ones,
                        n_pad, ncw_deg)
    p0 = degp[:n][:, None]
    p1 = degp[n_pad:n_pad + n][:, None]
    y = _y_tc(onehot, W, p0, p1, n, d, rows=1000)
    accp = _segsum_sc(src_p.reshape(NW * ncw_seg, K2),
                      dst_p.reshape(NW * ncw_seg, K2),
                      y, zeros2, n_pad, d, NW * ncw_seg)
    return _finish_tc(accp[:n], accp[n_pad:n_pad + n], y, p0, p1,
                      b.reshape(1, d), n, d, rows=1000)
